# baseline (speedup 1.0000x reference)
import jax
import jax.numpy as jnp
from jax.experimental import pallas as pl
from jax.experimental.pallas import tpu as pltpu

_D_IN = 512
_D_MODEL = 512
_N_HEADS = 4
_DH = _D_MODEL // _N_HEADS
_D_FF = 256
_N_LAYERS = 2
_K_SEL = 16
_L = 512
_SEQ = _L + 1
_SEQP = 520


def _nn(a, b):
    return jax.lax.dot_general(a, b, (((1,), (0,)), ((), ())),
                               preferred_element_type=jnp.float32)


def _nt(a, b):
    return jax.lax.dot_general(a, b, (((1,), (1,)), ((), ())),
                               preferred_element_type=jnp.float32)


def _nn_hi(a, b):
    return jax.lax.dot_general(a, b, (((1,), (0,)), ((), ())),
                               preferred_element_type=jnp.float32,
                               precision=jax.lax.Precision.HIGHEST)


def _tn_hi(a, b):
    return jax.lax.dot_general(a, b, (((0,), (0,)), ((), ())),
                               preferred_element_type=jnp.float32,
                               precision=jax.lax.Precision.HIGHEST)


def _halve8(r):
    r = r + pltpu.roll(r, 124, 1)
    r = r + pltpu.roll(r, 126, 1)
    r = r + pltpu.roll(r, 127, 1)
    return r[:, 0:1]


def _xla_row_sum(v, width):
    if width == 512:
        m = ((v[:, 0:128] + v[:, 128:256]) + v[:, 256:384]) + v[:, 384:512]
        acc = m
        cur = m
        for _ in range(15):
            cur = pltpu.roll(cur, 120, 1)
            acc = acc + cur
        return _halve8(acc)
    tail = jnp.concatenate(
        [v[:, 512:width],
         jnp.zeros((v.shape[0], 128 - (width - 512)), jnp.float32)], axis=1)
    vregs = [v[:, 0:128], v[:, 128:256], v[:, 256:384], v[:, 384:512], tail]
    acc = vregs[0]
    for j in range(5):
        vr = vregs[j]
        for i in range(1 if j == 0 else 0, 16):
            sh = (128 - 8 * i) % 128
            acc = acc + (vr if sh == 0 else pltpu.roll(vr, sh, 1))
    return _halve8(acc)


def _layer_norm(x, w, b):
    m = _xla_row_sum(x, _D_MODEL) * (1.0 / _D_MODEL)
    v = _xla_row_sum((x - m) ** 2, _D_MODEL) * (1.0 / _D_MODEL)
    return (x - m) * jax.lax.rsqrt(v + 1e-5) * w + b


def _encoder_topk_kernel(xv_ref, xt_ref, Wv_ref, bv_ref, Wt_ref, bt_ref,
                         me0_ref, me3_ref, Wqkv_ref, bqkv_ref, Wo_ref, bo_ref,
                         W1_ref, b1_ref, W2_ref, b2_ref, ln1w_ref, ln1b_ref,
                         ln2w_ref, ln2b_ref, wlg2_ref, blg_ref, out_ref):
    xv = xv_ref[0]
    xt = xt_ref[0]
    scale = jnp.sqrt(jnp.float32(_D_MODEL))
    v = (_nt(xv, Wv_ref[...]) + bv_ref[...]) * scale + me3_ref[...]
    t = (_nt(xt, Wt_ref[...]) + bt_ref[...]) * scale + me0_ref[...]
    pad = jnp.zeros((_SEQP - _SEQ, _D_MODEL), jnp.float32)
    x = jnp.concatenate([t, v, pad], axis=0)
    kcol = jax.lax.broadcasted_iota(jnp.int32, (_SEQP, _SEQP), 1)
    kmask = kcol < _SEQ
    for i in range(_N_LAYERS):
        qkv = _nt(x, Wqkv_ref[i]) + bqkv_ref[i]
        heads = []
        for h in range(_N_HEADS):
            q = qkv[:, h * _DH:(h + 1) * _DH]
            k = qkv[:, _D_MODEL + h * _DH:_D_MODEL + (h + 1) * _DH]
            vv = qkv[:, 2 * _D_MODEL + h * _DH:2 * _D_MODEL + (h + 1) * _DH]
            s = _nt(q, k) / jnp.sqrt(jnp.float32(_DH))
            s = jnp.where(kmask, s, -1e30)
            m = jnp.max(s, axis=-1, keepdims=True)
            e = jnp.exp(s - m)
            p = e / _xla_row_sum(e, _SEQP)
            heads.append(_nn(p, vv))
        o = jnp.concatenate(heads, axis=1)
        a = _nt(o, Wo_ref[i]) + bo_ref[i]
        x = _layer_norm(x + a, ln1w_ref[i], ln1b_ref[i])
        f = _nt(jax.nn.relu(_nt(x, W1_ref[i]) + b1_ref[i]), W2_ref[i]) + b2_ref[i]
        x = _layer_norm(x + f, ln2w_ref[i], ln2b_ref[i])

    x_b = x.astype(jnp.bfloat16).astype(jnp.float32)
    w_b = wlg2_ref[...][:, 0:1].astype(jnp.bfloat16).astype(jnp.float32)
    prod = x_b * _tn_hi(w_b, jnp.where(jax.lax.broadcasted_iota(jnp.int32, (_D_MODEL, _D_MODEL), 0) == jax.lax.broadcasted_iota(jnp.int32, (_D_MODEL, _D_MODEL), 1), 1.0, 0.0))
    s_all = _xla_row_sum(prod, _D_MODEL) + blg_ref[0, 0]
    rr = jax.lax.broadcasted_iota(jnp.int32, (_SEQP, 1), 0)
    is_frame = (rr >= 1) & (rr <= _L)
    s_col = jnp.where(is_frame, s_all, -3e38)
    ii = jax.lax.broadcasted_iota(jnp.int32, (_SEQP, _SEQP), 0)
    jj = jax.lax.broadcasted_iota(jnp.int32, (_SEQP, _SEQP), 1)
    ident = jnp.where(ii == jj, 1.0, 0.0)
    s_row = _tn_hi(s_col, ident)
    beats = (s_row > s_col) | ((s_row == s_col) & (jj < ii))
    rank = jnp.sum(jnp.where(beats, 1.0, 0.0), axis=1, keepdims=True)
    sel = rank < jnp.float32(_K_SEL)
    lower = jnp.where(jj < ii, 1.0, 0.0)
    cnt = _nn_hi(lower, jnp.where(sel, 1.0, 0.0))
    pf = jax.lax.broadcasted_iota(jnp.int32, (_SEQP, _K_SEL), 1).astype(jnp.float32)
    onehot = jnp.where((cnt == pf) & sel, 1.0, 0.0)
    xv_pad = jnp.concatenate(
        [jnp.zeros((1, _D_IN), jnp.float32), xv,
         jnp.zeros((_SEQP - _SEQ, _D_IN), jnp.float32)], axis=0)
    out_ref[0, :, :] = _tn_hi(onehot, xv_pad)


def kernel(x_vis_seq, x_txt_query, Wv, bv, Wt, bt, mod_emb, Wqkv, bqkv, Wo,
           bo, W1, b1, W2, b2, ln1w, ln1b, ln2w, ln2b, Wlg, blg):
    n = x_vis_seq.shape[0]
    xt3 = x_txt_query[:, None, :]
    wlg2 = jnp.concatenate(
        [Wlg.T, jnp.zeros((_D_MODEL, 127), jnp.float32)], axis=1)
    full = lambda *shape: pl.BlockSpec(shape, lambda b: (0,) * len(shape))
    in_specs = [
        pl.BlockSpec((1, _L, _D_IN), lambda b: (b, 0, 0)),
        pl.BlockSpec((1, 1, _D_IN), lambda b: (b, 0, 0)),
        full(_D_MODEL, _D_IN),
        full(1, _D_MODEL),
        full(_D_MODEL, _D_IN),
        full(1, _D_MODEL),
        full(1, _D_MODEL),
        full(1, _D_MODEL),
        full(_N_LAYERS, 3 * _D_MODEL, _D_MODEL),
        full(_N_LAYERS, 1, 3 * _D_MODEL),
        full(_N_LAYERS, _D_MODEL, _D_MODEL),
        full(_N_LAYERS, 1, _D_MODEL),
        full(_N_LAYERS, _D_FF, _D_MODEL),
        full(_N_LAYERS, 1, _D_FF),
        full(_N_LAYERS, _D_MODEL, _D_FF),
        full(_N_LAYERS, 1, _D_MODEL),
        full(_N_LAYERS, 1, _D_MODEL),
        full(_N_LAYERS, 1, _D_MODEL),
        full(_N_LAYERS, 1, _D_MODEL),
        full(_N_LAYERS, 1, _D_MODEL),
        full(_D_MODEL, 128),
        full(1, 1),
    ]
    out = pl.pallas_call(
        _encoder_topk_kernel,
        grid=(n,),
        in_specs=in_specs,
        out_specs=pl.BlockSpec((1, _K_SEL, _D_IN), lambda b: (b, 0, 0)),
        out_shape=jax.ShapeDtypeStruct((n, _K_SEL, _D_IN), jnp.float32),
        compiler_params=pltpu.CompilerParams(
            dimension_semantics=("parallel",)),
    )(x_vis_seq, xt3, Wv, bv[None, :], Wt, bt[None, :],
      mod_emb[0][None, :], mod_emb[3][None, :],
      Wqkv, bqkv[:, None, :], Wo, bo[:, None, :],
      W1, b1[:, None, :], W2, b2[:, None, :],
      ln1w[:, None, :], ln1b[:, None, :], ln2w[:, None, :], ln2b[:, None, :],
      wlg2, blg[None, :])
    return out
